# baseline (speedup 1.0000x reference)
import functools

import jax
import jax.numpy as jnp
from jax import lax
from jax.experimental import pallas as pl
from jax.experimental.pallas import tpu as pltpu
from jax.experimental.pallas import tpu_sc as plsc

N_FIELDS = 26
VOCAB = 100000
EMB = 16
OHE = 100
HID = 128
BATCH = 16384
EMB_FEAT = N_FIELDS * EMB
NPLANE = 4

NC, NS = 2, 16
NW = NC * NS
BATCH_PER_W = BATCH // NW
SUPER = 64
NSUPER = BATCH_PER_W // SUPER


def _sc_gather_body(idx_hbm, tab_hbm, out_hbm, idxt_v, stage1, stage2, sem):
    wid = lax.axis_index("s") * NC + lax.axis_index("c")
    wb0 = wid * BATCH_PER_W
    pltpu.sync_copy(idx_hbm.at[:, pl.ds(wb0, BATCH_PER_W)], idxt_v)

    zeros = jnp.zeros((EMB,), jnp.float32)

    def superchunk(s, _):
        b0 = s * SUPER

        def g_copy(f):
            return pltpu.make_async_copy(
                tab_hbm.at[f].at[idxt_v.at[f, pl.ds(b0, SUPER)]],
                stage1.at[f], sem)

        def fire(f, _):
            g_copy(f).start()
            return _

        lax.fori_loop(0, N_FIELDS, fire, None)

        def drain(f, _):
            g_copy(f).wait()
            return _

        lax.fori_loop(0, N_FIELDS, drain, None)

        def repack(bl, _):
            for p in range(NPLANE):
                for e in range(8):
                    f = 8 * p + e
                    if f < N_FIELDS:
                        stage2[p, bl, pl.ds(EMB * e, EMB)] = stage1[f, bl, :]
                    else:
                        stage2[p, bl, pl.ds(EMB * e, EMB)] = zeros
            return _

        lax.fori_loop(0, SUPER, repack, None)

        for p in range(NPLANE):
            pltpu.sync_copy(stage2.at[p],
                            out_hbm.at[p, pl.ds(wb0 + b0, SUPER)])
        return _

    lax.fori_loop(0, NSUPER, superchunk, None)


_sc_gather = pl.kernel(
    _sc_gather_body,
    out_type=jax.ShapeDtypeStruct((NPLANE, BATCH, HID), jnp.float32),
    mesh=plsc.VectorSubcoreMesh(core_axis_name="c", subcore_axis_name="s"),
    compiler_params=pltpu.CompilerParams(use_tc_tiling_on_sc=False),
    scratch_types=[
        pltpu.VMEM((N_FIELDS, BATCH_PER_W), jnp.int32),
        pltpu.VMEM((N_FIELDS, SUPER, EMB), jnp.float32),
        pltpu.VMEM((NPLANE, SUPER, HID), jnp.float32),
        pltpu.SemaphoreType.DMA,
    ],
)


def _tr_body(idx_ref, out_ref):
    out_ref[...] = idx_ref[...].T


def _transpose_idx(embed_idx):
    bm = 2048
    return pl.pallas_call(
        _tr_body,
        grid=(BATCH // bm,),
        in_specs=[pl.BlockSpec((bm, N_FIELDS), lambda m: (m, 0))],
        out_specs=pl.BlockSpec((N_FIELDS, bm), lambda m: (0, m)),
        out_shape=jax.ShapeDtypeStruct((N_FIELDS, BATCH), jnp.int32),
    )(embed_idx)


def _mm_body(g_ref, o_ref, w1_ref, w2_ref, b_ref, out_ref):
    acc = jnp.dot(o_ref[...], w2_ref[...], preferred_element_type=jnp.float32)
    for p in range(NPLANE):
        acc += jnp.dot(g_ref[p], w1_ref[p],
                       preferred_element_type=jnp.float32)
    out_ref[...] = acc + b_ref[...]


def _dense(planes, ohes, w1, w2, b2):
    bm = 1024
    return pl.pallas_call(
        _mm_body,
        grid=(BATCH // bm,),
        in_specs=[
            pl.BlockSpec((NPLANE, bm, HID), lambda m: (0, m, 0)),
            pl.BlockSpec((bm, OHE), lambda m: (m, 0)),
            pl.BlockSpec((NPLANE, HID, HID), lambda m: (0, 0, 0)),
            pl.BlockSpec((OHE, HID), lambda m: (0, 0)),
            pl.BlockSpec((1, HID), lambda m: (0, 0)),
        ],
        out_specs=pl.BlockSpec((bm, HID), lambda m: (m, 0)),
        out_shape=jax.ShapeDtypeStruct((BATCH, HID), jnp.float32),
    )(planes, ohes, w1, w2, b2)


def kernel(embed_idx, ohes, tables, W, b):
    idx_t = _transpose_idx(embed_idx.astype(jnp.int32))
    planes = _sc_gather(idx_t, tables)
    w1 = jnp.pad(W[:EMB_FEAT], ((0, NPLANE * HID - EMB_FEAT), (0, 0)))
    w1 = w1.reshape(NPLANE, HID, HID)
    return _dense(planes, ohes, w1, W[EMB_FEAT:], b.reshape(1, HID))


# speedup vs baseline: 1.0358x; 1.0358x over previous
import functools

import jax
import jax.numpy as jnp
from jax import lax
from jax.experimental import pallas as pl
from jax.experimental.pallas import tpu as pltpu
from jax.experimental.pallas import tpu_sc as plsc

N_FIELDS = 26
VOCAB = 100000
EMB = 16
OHE = 100
HID = 128
BATCH = 16384
EMB_FEAT = N_FIELDS * EMB
NPLANE = 4

NC, NS = 2, 16
NW = NC * NS
BATCH_PER_W = BATCH // NW
TOT_ROWS = BATCH * N_FIELDS
ROWS_PER_W = TOT_ROWS // NW

PACK = 8
TAB_ROWS = N_FIELDS * VOCAB // PACK

A_TILES = 32
A_VROWS = A_TILES * PACK
TOT_VROWS = N_FIELDS * VOCAB
A_UNITS = (TOT_VROWS + A_VROWS - 1) // A_VROWS
A_KMAX = (A_UNITS + NW - 1) // NW

CHUNK = 104
CPW = ROWS_PER_W // CHUNK
SUPER = 64
CPS = SUPER * N_FIELDS // CHUNK
NSUPER = BATCH_PER_W // SUPER


def _depad_body(tab_hbm, out_hbm, src_v, out_v, sem0, sem1):
    wid = lax.axis_index("s") * NC + lax.axis_index("c")
    sems = (sem0, sem1)

    def unit(k):
        return wid + k * NW

    def vstart(k):
        s = jnp.minimum(unit(k) * A_VROWS, TOT_VROWS - A_VROWS)
        return pl.multiple_of(s, PACK)

    def a_copy(k, buf):
        return pltpu.make_async_copy(
            tab_hbm.at[pl.ds(vstart(k), A_VROWS)], src_v.at[buf], sems[buf])

    @pl.when(unit(0) < A_UNITS)
    def _():
        a_copy(0, 0).start()

    def pair_body(pk, _):
        for h in range(2):
            k = pk * 2 + h

            @pl.when(unit(k) < A_UNITS)
            def _do():
                a_copy(k, h).wait()

                @pl.when(unit(k + 1) < A_UNITS)
                def _fire():
                    a_copy(k + 1, 1 - h).start()

                def repack(t, _):
                    for j in range(PACK):
                        out_v[t, pl.ds(EMB * j, EMB)] = src_v[
                            h, t * PACK + j, :]
                    return _

                lax.fori_loop(0, A_TILES, repack, None)

                pltpu.sync_copy(
                    out_v,
                    out_hbm.at[pl.ds(
                        pl.multiple_of(lax.div(vstart(k), PACK), PACK),
                        A_TILES)])
        return _

    lax.fori_loop(0, (A_KMAX + 1) // 2, pair_body, None)


_depad = pl.kernel(
    _depad_body,
    out_type=jax.ShapeDtypeStruct((TAB_ROWS, HID), jnp.float32),
    mesh=plsc.VectorSubcoreMesh(core_axis_name="c", subcore_axis_name="s"),
    compiler_params=pltpu.CompilerParams(use_tc_tiling_on_sc=True),
    scratch_types=[
        pltpu.VMEM((2, A_VROWS, EMB), jnp.float32),
        pltpu.VMEM((A_TILES, HID), jnp.float32),
        pltpu.SemaphoreType.DMA,
        pltpu.SemaphoreType.DMA,
    ],
)

_GROUPS = [(16 * g, range(16)) for g in range(6)] + [(88, range(8, 16))]


def _gather_body(q_hbm, s_hbm, tab_hbm, out_hbm, q_v, s_v, stage1, stage2,
                 sem0, sem1):
    wid = lax.axis_index("s") * NC + lax.axis_index("c")
    sems = (sem0, sem1)
    pltpu.sync_copy(q_hbm.at[pl.ds(wid * CPW, CPW)], q_v)
    pltpu.sync_copy(s_hbm.at[pl.ds(wid * ROWS_PER_W, ROWS_PER_W)], s_v)

    zeros = jnp.zeros((EMB,), jnp.float32)

    def zero_body(bl, _):
        for k in range(6):
            stage2[3, bl, pl.ds(32 + EMB * k, EMB)] = zeros
        return _

    lax.fori_loop(0, SUPER, zero_body, None)

    def g_copy(cw, buf):
        return pltpu.make_async_copy(
            tab_hbm.at[q_v.at[cw]], stage1.at[buf], sems[buf])

    def superchunk(sc, _):
        g_copy(sc * CPS, 0).start()

        def pair_body(pk, _):
            for h in range(2):
                cl = pk * 2 + h
                cw = sc * CPS + cl
                g_copy(cw, h).wait()

                @pl.when(cl < CPS - 1)
                def _fire():
                    g_copy(cw + 1, 1 - h).start()

                for base, lanes in _GROUPS:
                    svec = s_v[pl.ds(cw * CHUNK + base, 16)]
                    for k in lanes:
                        i = base + k
                        f = i % N_FIELDS
                        stage2[f // 8, cl * 4 + i // N_FIELDS,
                               pl.ds(EMB * (f % 8), EMB)] = stage1[
                            h, i, pl.ds(svec[k] * EMB, EMB)]
            return _

        lax.fori_loop(0, CPS // 2, pair_body, None)

        for p in range(NPLANE):
            pltpu.sync_copy(
                stage2.at[p],
                out_hbm.at[p, pl.ds(wid * BATCH_PER_W + sc * SUPER, SUPER)])
        return _

    lax.fori_loop(0, NSUPER, superchunk, None)


_sc_gather = pl.kernel(
    _gather_body,
    out_type=jax.ShapeDtypeStruct((NPLANE, BATCH, HID), jnp.float32),
    mesh=plsc.VectorSubcoreMesh(core_axis_name="c", subcore_axis_name="s"),
    compiler_params=pltpu.CompilerParams(use_tc_tiling_on_sc=False),
    scratch_types=[
        pltpu.VMEM((CPW * NW // NW, CHUNK), jnp.int32),
        pltpu.VMEM((ROWS_PER_W,), jnp.int32),
        pltpu.VMEM((2, CHUNK, HID), jnp.float32),
        pltpu.VMEM((NPLANE, SUPER, HID), jnp.float32),
        pltpu.SemaphoreType.DMA,
        pltpu.SemaphoreType.DMA,
    ],
)


def _mm_body(g_ref, o_ref, w1_ref, w2_ref, b_ref, out_ref):
    acc = jnp.dot(o_ref[...], w2_ref[...], preferred_element_type=jnp.float32)
    for p in range(NPLANE):
        acc += jnp.dot(g_ref[p], w1_ref[p],
                       preferred_element_type=jnp.float32)
    out_ref[...] = acc + b_ref[...]


def _dense(planes, ohes, w1, w2, b2):
    bm = 1024
    return pl.pallas_call(
        _mm_body,
        grid=(BATCH // bm,),
        in_specs=[
            pl.BlockSpec((NPLANE, bm, HID), lambda m: (0, m, 0)),
            pl.BlockSpec((bm, OHE), lambda m: (m, 0)),
            pl.BlockSpec((NPLANE, HID, HID), lambda m: (0, 0, 0)),
            pl.BlockSpec((OHE, HID), lambda m: (0, 0)),
            pl.BlockSpec((1, HID), lambda m: (0, 0)),
        ],
        out_specs=pl.BlockSpec((bm, HID), lambda m: (m, 0)),
        out_shape=jax.ShapeDtypeStruct((BATCH, HID), jnp.float32),
    )(planes, ohes, w1, w2, b2)


def kernel(embed_idx, ohes, tables, W, b):
    tab128 = _depad(tables.reshape(TOT_VROWS, EMB))
    offs = (jnp.arange(N_FIELDS, dtype=jnp.int32) * VOCAB)[None, :]
    r = embed_idx.astype(jnp.int32) + offs
    q2d = (r // PACK).reshape(TOT_ROWS // CHUNK, CHUNK)
    s1d = (r % PACK).reshape(TOT_ROWS)
    planes = _sc_gather(q2d, s1d, tab128)
    w1 = jnp.pad(W[:EMB_FEAT], ((0, NPLANE * HID - EMB_FEAT), (0, 0)))
    w1 = w1.reshape(NPLANE, HID, HID)
    return _dense(planes, ohes, w1, W[EMB_FEAT:], b.reshape(1, HID))


# speedup vs baseline: 1.1224x; 1.0836x over previous
import functools

import jax
import jax.numpy as jnp
from jax import lax
from jax.experimental import pallas as pl
from jax.experimental.pallas import tpu as pltpu
from jax.experimental.pallas import tpu_sc as plsc

N_FIELDS = 26
VOCAB = 100000
EMB = 16
OHE = 100
HID = 128
BATCH = 16384
EMB_FEAT = N_FIELDS * EMB
NPLANE = 4

NC, NS = 2, 16
NW = NC * NS
BATCH_PER_W = BATCH // NW
TOT_ROWS = BATCH * N_FIELDS
ROWS_PER_W = TOT_ROWS // NW

PACK = 8
TAB_ROWS = N_FIELDS * VOCAB // PACK

A_TILES = 56
A_VROWS = A_TILES * PACK
TOT_VROWS = N_FIELDS * VOCAB
A_UNITS = (TOT_VROWS + A_VROWS - 1) // A_VROWS
A_KMAX = (A_UNITS + NW - 1) // NW

CHUNK = 104
CPW = ROWS_PER_W // CHUNK
SUPER = 64
CPS = SUPER * N_FIELDS // CHUNK
NSUPER = BATCH_PER_W // SUPER


def _depad_body(tab_hbm, out_hbm, src_v, out_v, isem0, isem1, osem0, osem1):
    wid = lax.axis_index("s") * NC + lax.axis_index("c")
    isems = (isem0, isem1)
    osems = (osem0, osem1)
    kmax_w = lax.div(A_UNITS - wid + NW - 1, NW)

    def unit(k):
        return wid + k * NW

    def vstart(k):
        s = jnp.minimum(unit(k) * A_VROWS, TOT_VROWS - A_VROWS)
        return pl.multiple_of(s, PACK)

    def a_copy(k, buf):
        return pltpu.make_async_copy(
            tab_hbm.at[pl.ds(vstart(k), A_VROWS)], src_v.at[buf],
            isems[buf])

    def o_copy(k, buf):
        return pltpu.make_async_copy(
            out_v.at[buf],
            out_hbm.at[pl.ds(
                pl.multiple_of(lax.div(vstart(k), PACK), PACK), A_TILES)],
            osems[buf])

    @pl.when(unit(0) < A_UNITS)
    def _():
        a_copy(0, 0).start()

    def pair_body(pk, _):
        for h in range(2):
            k = pk * 2 + h

            @pl.when(unit(k) < A_UNITS)
            def _do():
                a_copy(k, h).wait()

                @pl.when(unit(k + 1) < A_UNITS)
                def _fire():
                    a_copy(k + 1, 1 - h).start()

                @pl.when(k >= 2)
                def _drain_out():
                    o_copy(k, h).wait()

                def repack(t, _):
                    for j in range(PACK):
                        out_v[h, t, pl.ds(EMB * j, EMB)] = src_v[
                            h, t * PACK + j, :]
                    return _

                lax.fori_loop(0, A_TILES, repack, None)
                o_copy(k, h).start()
        return _

    lax.fori_loop(0, A_KMAX // 2, pair_body, None)

    for h in range(2):
        @pl.when(kmax_w >= h + 1)
        def _final_drain():
            o_copy(0, h).wait()


_depad = pl.kernel(
    _depad_body,
    out_type=jax.ShapeDtypeStruct((TAB_ROWS, HID), jnp.float32),
    mesh=plsc.VectorSubcoreMesh(core_axis_name="c", subcore_axis_name="s"),
    compiler_params=pltpu.CompilerParams(use_tc_tiling_on_sc=True),
    scratch_types=[
        pltpu.VMEM((2, A_VROWS, EMB), jnp.float32),
        pltpu.VMEM((2, A_TILES, HID), jnp.float32),
        pltpu.SemaphoreType.DMA,
        pltpu.SemaphoreType.DMA,
        pltpu.SemaphoreType.DMA,
        pltpu.SemaphoreType.DMA,
    ],
)

_GROUPS = [(16 * g, range(16)) for g in range(6)] + [(88, range(8, 16))]


def _gather_body(q_hbm, s_hbm, tab_hbm, out_hbm, q_v, s_v, stage1, stage2,
                 sem0, sem1):
    wid = lax.axis_index("s") * NC + lax.axis_index("c")
    sems = (sem0, sem1)
    pltpu.sync_copy(q_hbm.at[pl.ds(wid * CPW, CPW)], q_v)
    pltpu.sync_copy(s_hbm.at[pl.ds(wid * ROWS_PER_W, ROWS_PER_W)], s_v)

    zeros = jnp.zeros((EMB,), jnp.float32)

    def zero_body(bl, _):
        for k in range(6):
            stage2[3, bl, pl.ds(32 + EMB * k, EMB)] = zeros
        return _

    lax.fori_loop(0, SUPER, zero_body, None)

    def g_copy(cw, buf):
        return pltpu.make_async_copy(
            tab_hbm.at[q_v.at[cw]], stage1.at[buf], sems[buf])

    def superchunk(sc, _):
        g_copy(sc * CPS, 0).start()

        def pair_body(pk, _):
            for h in range(2):
                cl = pk * 2 + h
                cw = sc * CPS + cl
                g_copy(cw, h).wait()

                @pl.when(cl < CPS - 1)
                def _fire():
                    g_copy(cw + 1, 1 - h).start()

                for base, lanes in _GROUPS:
                    svec = s_v[pl.ds(cw * CHUNK + base, 16)]
                    for k in lanes:
                        i = base + k
                        f = i % N_FIELDS
                        stage2[f // 8, cl * 4 + i // N_FIELDS,
                               pl.ds(EMB * (f % 8), EMB)] = stage1[
                            h, i, pl.ds(svec[k] * EMB, EMB)]
            return _

        lax.fori_loop(0, CPS // 2, pair_body, None)

        for p in range(NPLANE):
            pltpu.sync_copy(
                stage2.at[p],
                out_hbm.at[p, pl.ds(wid * BATCH_PER_W + sc * SUPER, SUPER)])
        return _

    lax.fori_loop(0, NSUPER, superchunk, None)


_sc_gather = pl.kernel(
    _gather_body,
    out_type=jax.ShapeDtypeStruct((NPLANE, BATCH, HID), jnp.float32),
    mesh=plsc.VectorSubcoreMesh(core_axis_name="c", subcore_axis_name="s"),
    compiler_params=pltpu.CompilerParams(use_tc_tiling_on_sc=False),
    scratch_types=[
        pltpu.VMEM((CPW * NW // NW, CHUNK), jnp.int32),
        pltpu.VMEM((ROWS_PER_W,), jnp.int32),
        pltpu.VMEM((2, CHUNK, HID), jnp.float32),
        pltpu.VMEM((NPLANE, SUPER, HID), jnp.float32),
        pltpu.SemaphoreType.DMA,
        pltpu.SemaphoreType.DMA,
    ],
)


def _mm_body(g_ref, o_ref, w1_ref, w2_ref, b_ref, out_ref):
    acc = jnp.dot(o_ref[...], w2_ref[...], preferred_element_type=jnp.float32)
    for p in range(NPLANE):
        acc += jnp.dot(g_ref[p], w1_ref[p],
                       preferred_element_type=jnp.float32)
    out_ref[...] = acc + b_ref[...]


def _dense(planes, ohes, w1, w2, b2):
    bm = 1024
    return pl.pallas_call(
        _mm_body,
        grid=(BATCH // bm,),
        in_specs=[
            pl.BlockSpec((NPLANE, bm, HID), lambda m: (0, m, 0)),
            pl.BlockSpec((bm, OHE), lambda m: (m, 0)),
            pl.BlockSpec((NPLANE, HID, HID), lambda m: (0, 0, 0)),
            pl.BlockSpec((OHE, HID), lambda m: (0, 0)),
            pl.BlockSpec((1, HID), lambda m: (0, 0)),
        ],
        out_specs=pl.BlockSpec((bm, HID), lambda m: (m, 0)),
        out_shape=jax.ShapeDtypeStruct((BATCH, HID), jnp.float32),
    )(planes, ohes, w1, w2, b2)


def kernel(embed_idx, ohes, tables, W, b):
    tab128 = _depad(tables.reshape(TOT_VROWS, EMB))
    offs = (jnp.arange(N_FIELDS, dtype=jnp.int32) * VOCAB)[None, :]
    r = embed_idx.astype(jnp.int32) + offs
    q2d = (r // PACK).reshape(TOT_ROWS // CHUNK, CHUNK)
    s1d = (r % PACK).reshape(TOT_ROWS)
    planes = _sc_gather(q2d, s1d, tab128)
    w1 = jnp.pad(W[:EMB_FEAT], ((0, NPLANE * HID - EMB_FEAT), (0, 0)))
    w1 = w1.reshape(NPLANE, HID, HID)
    return _dense(planes, ohes, w1, W[EMB_FEAT:], b.reshape(1, HID))


# speedup vs baseline: 1.2060x; 1.0744x over previous
import functools

import jax
import jax.numpy as jnp
from jax import lax
from jax.experimental import pallas as pl
from jax.experimental.pallas import tpu as pltpu
from jax.experimental.pallas import tpu_sc as plsc

N_FIELDS = 26
VOCAB = 100000
EMB = 16
OHE = 100
HID = 128
BATCH = 16384
EMB_FEAT = N_FIELDS * EMB
NPLANE = 4

NC, NS = 2, 16
NW = NC * NS
BATCH_PER_W = BATCH // NW
TOT_ROWS = BATCH * N_FIELDS
ROWS_PER_W = TOT_ROWS // NW

PACK = 8
TAB_ROWS = N_FIELDS * VOCAB // PACK

A_TILES = 56
A_VROWS = A_TILES * PACK
TOT_VROWS = N_FIELDS * VOCAB
A_UNITS = (TOT_VROWS + A_VROWS - 1) // A_VROWS
A_KMAX = (A_UNITS + NW - 1) // NW

CHUNK = 104
CPW = ROWS_PER_W // CHUNK
SUPER = 64
CPS = SUPER * N_FIELDS // CHUNK
NSUPER = BATCH_PER_W // SUPER


def _tcpack_body(in_ref, out_ref):
    out_ref[...] = in_ref[...].reshape(out_ref.shape)


def _tc_pack(tab3d):
    bm = 2600
    return pl.pallas_call(
        _tcpack_body,
        grid=(TAB_ROWS // bm,),
        in_specs=[pl.BlockSpec((bm, PACK, EMB), lambda m: (m, 0, 0))],
        out_specs=pl.BlockSpec((bm, HID), lambda m: (m, 0)),
        out_shape=jax.ShapeDtypeStruct((TAB_ROWS, HID), jnp.float32),
    )(tab3d)


def _depad_body(tab_hbm, out_hbm, src_v, out_v, isem0, isem1, osem0, osem1):
    wid = lax.axis_index("s") * NC + lax.axis_index("c")
    isems = (isem0, isem1)
    osems = (osem0, osem1)
    kmax_w = lax.div(A_UNITS - wid + NW - 1, NW)

    def unit(k):
        return wid + k * NW

    def vstart(k):
        s = jnp.minimum(unit(k) * A_VROWS, TOT_VROWS - A_VROWS)
        return pl.multiple_of(s, PACK)

    def a_copy(k, buf):
        return pltpu.make_async_copy(
            tab_hbm.at[pl.ds(vstart(k), A_VROWS)], src_v.at[buf],
            isems[buf])

    def o_copy(k, buf):
        return pltpu.make_async_copy(
            out_v.at[buf],
            out_hbm.at[pl.ds(
                pl.multiple_of(lax.div(vstart(k), PACK), PACK), A_TILES)],
            osems[buf])

    @pl.when(unit(0) < A_UNITS)
    def _():
        a_copy(0, 0).start()

    def pair_body(pk, _):
        for h in range(2):
            k = pk * 2 + h

            @pl.when(unit(k) < A_UNITS)
            def _do():
                a_copy(k, h).wait()

                @pl.when(unit(k + 1) < A_UNITS)
                def _fire():
                    a_copy(k + 1, 1 - h).start()

                @pl.when(k >= 2)
                def _drain_out():
                    o_copy(k, h).wait()

                def repack(t, _):
                    for j in range(PACK):
                        out_v[h, t, pl.ds(EMB * j, EMB)] = src_v[
                            h, t * PACK + j, :]
                    return _

                lax.fori_loop(0, A_TILES, repack, None)
                o_copy(k, h).start()
        return _

    lax.fori_loop(0, A_KMAX // 2, pair_body, None)

    for h in range(2):
        @pl.when(kmax_w >= h + 1)
        def _final_drain():
            o_copy(0, h).wait()


_depad = pl.kernel(
    _depad_body,
    out_type=jax.ShapeDtypeStruct((TAB_ROWS, HID), jnp.float32),
    mesh=plsc.VectorSubcoreMesh(core_axis_name="c", subcore_axis_name="s"),
    compiler_params=pltpu.CompilerParams(use_tc_tiling_on_sc=True),
    scratch_types=[
        pltpu.VMEM((2, A_VROWS, EMB), jnp.float32),
        pltpu.VMEM((2, A_TILES, HID), jnp.float32),
        pltpu.SemaphoreType.DMA,
        pltpu.SemaphoreType.DMA,
        pltpu.SemaphoreType.DMA,
        pltpu.SemaphoreType.DMA,
    ],
)

_GROUPS = [(16 * g, range(16)) for g in range(6)] + [(88, range(8, 16))]


def _gather_body(q_hbm, s_hbm, tab_hbm, out_hbm, q_v, s_v, stage1, stage2,
                 sem0, sem1):
    wid = lax.axis_index("s") * NC + lax.axis_index("c")
    sems = (sem0, sem1)
    pltpu.sync_copy(q_hbm.at[pl.ds(wid * CPW, CPW)], q_v)
    pltpu.sync_copy(s_hbm.at[pl.ds(wid * ROWS_PER_W, ROWS_PER_W)], s_v)

    zeros = jnp.zeros((EMB,), jnp.float32)

    def zero_body(bl, _):
        for k in range(6):
            stage2[3, bl, pl.ds(32 + EMB * k, EMB)] = zeros
        return _

    lax.fori_loop(0, SUPER, zero_body, None)

    def g_copy(cw, buf):
        return pltpu.make_async_copy(
            tab_hbm.at[q_v.at[cw]], stage1.at[buf], sems[buf])

    def superchunk(sc, _):
        g_copy(sc * CPS, 0).start()

        def pair_body(pk, _):
            for h in range(2):
                cl = pk * 2 + h
                cw = sc * CPS + cl
                g_copy(cw, h).wait()

                @pl.when(cl < CPS - 1)
                def _fire():
                    g_copy(cw + 1, 1 - h).start()

                for base, lanes in _GROUPS:
                    svec = s_v[pl.ds(cw * CHUNK + base, 16)]
                    for k in lanes:
                        i = base + k
                        f = i % N_FIELDS
                        stage2[f // 8, cl * 4 + i // N_FIELDS,
                               pl.ds(EMB * (f % 8), EMB)] = stage1[
                            h, i, pl.ds(svec[k] * EMB, EMB)]
            return _

        lax.fori_loop(0, CPS // 2, pair_body, None)

        for p in range(NPLANE):
            pltpu.sync_copy(
                stage2.at[p],
                out_hbm.at[p, pl.ds(wid * BATCH_PER_W + sc * SUPER, SUPER)])
        return _

    lax.fori_loop(0, NSUPER, superchunk, None)


_sc_gather = pl.kernel(
    _gather_body,
    out_type=jax.ShapeDtypeStruct((NPLANE, BATCH, HID), jnp.float32),
    mesh=plsc.VectorSubcoreMesh(core_axis_name="c", subcore_axis_name="s"),
    compiler_params=pltpu.CompilerParams(use_tc_tiling_on_sc=False),
    scratch_types=[
        pltpu.VMEM((CPW * NW // NW, CHUNK), jnp.int32),
        pltpu.VMEM((ROWS_PER_W,), jnp.int32),
        pltpu.VMEM((2, CHUNK, HID), jnp.float32),
        pltpu.VMEM((NPLANE, SUPER, HID), jnp.float32),
        pltpu.SemaphoreType.DMA,
        pltpu.SemaphoreType.DMA,
    ],
)


def _mm_body(g_ref, o_ref, w1_ref, w2_ref, b_ref, out_ref):
    acc = jnp.dot(o_ref[...], w2_ref[...], preferred_element_type=jnp.float32)
    for p in range(NPLANE):
        acc += jnp.dot(g_ref[p], w1_ref[p],
                       preferred_element_type=jnp.float32)
    out_ref[...] = acc + b_ref[...]


def _dense(planes, ohes, w1, w2, b2):
    bm = 1024
    return pl.pallas_call(
        _mm_body,
        grid=(BATCH // bm,),
        in_specs=[
            pl.BlockSpec((NPLANE, bm, HID), lambda m: (0, m, 0)),
            pl.BlockSpec((bm, OHE), lambda m: (m, 0)),
            pl.BlockSpec((NPLANE, HID, HID), lambda m: (0, 0, 0)),
            pl.BlockSpec((OHE, HID), lambda m: (0, 0)),
            pl.BlockSpec((1, HID), lambda m: (0, 0)),
        ],
        out_specs=pl.BlockSpec((bm, HID), lambda m: (m, 0)),
        out_shape=jax.ShapeDtypeStruct((BATCH, HID), jnp.float32),
    )(planes, ohes, w1, w2, b2)


def kernel(embed_idx, ohes, tables, W, b):
    tab128 = _tc_pack(tables.reshape(TAB_ROWS, PACK, EMB))
    offs = (jnp.arange(N_FIELDS, dtype=jnp.int32) * VOCAB)[None, :]
    r = embed_idx.astype(jnp.int32) + offs
    q2d = (r // PACK).reshape(TOT_ROWS // CHUNK, CHUNK)
    s1d = (r % PACK).reshape(TOT_ROWS)
    planes = _sc_gather(q2d, s1d, tab128)
    w1 = jnp.pad(W[:EMB_FEAT], ((0, NPLANE * HID - EMB_FEAT), (0, 0)))
    w1 = w1.reshape(NPLANE, HID, HID)
    return _dense(planes, ohes, w1, W[EMB_FEAT:], b.reshape(1, HID))
